# baseline (speedup 1.0000x reference)
import jax
import jax.numpy as jnp
from jax.experimental import pallas as pl

_BR = 512


def _add_body(x_ref, p_ref, o_ref):
    i = pl.program_id(0)
    nblk = p_ref.shape[0] // _BR
    o_ref[...] = x_ref[...] + p_ref[pl.ds((i % nblk) * _BR, _BR), :]


def kernel(x, pos_table):
    T, D = pos_table.shape
    xf = x.reshape(-1, D)
    N = xf.shape[0]
    grid = (N // _BR,)
    out = pl.pallas_call(
        _add_body,
        grid=grid,
        in_specs=[
            pl.BlockSpec((_BR, D), lambda i: (i, 0)),
            pl.BlockSpec((T, D), lambda i: (0, 0)),
        ],
        out_specs=pl.BlockSpec((_BR, D), lambda i: (i, 0)),
        out_shape=jax.ShapeDtypeStruct((N, D), x.dtype),
    )(xf, pos_table)
    return out.reshape(-1, T, D)


# speedup vs baseline: 1.2030x; 1.2030x over previous
import jax
import jax.numpy as jnp
from jax.experimental import pallas as pl


def _add_body(x_ref, p_ref, o_ref):
    o_ref[...] = x_ref[...] + p_ref[...]


def kernel(x, pos_table):
    T, D = pos_table.shape
    xf = x.reshape(-1, D)
    N = xf.shape[0]
    grid = (N // T,)
    out = pl.pallas_call(
        _add_body,
        grid=grid,
        in_specs=[
            pl.BlockSpec((T, D), lambda i: (i, 0)),
            pl.BlockSpec((T, D), lambda i: (0, 0)),
        ],
        out_specs=pl.BlockSpec((T, D), lambda i: (i, 0)),
        out_shape=jax.ShapeDtypeStruct((N, D), x.dtype),
    )(xf, pos_table)
    return out.reshape(-1, T, D)
